# baseline (speedup 1.0000x reference)
import functools

import jax
import jax.numpy as jnp
from jax import lax
from jax.experimental import pallas as pl
from jax.experimental.pallas import tpu as pltpu
from jax.experimental.pallas import tpu_sc as plsc

N = 10000
NP = 10240
D = 256
HH = 128
E = 160000
EP = 163840
DUMMY = 10000
NC = 2
NS = 16
B = 128
R = 512
GRID = NP // R
ROWS_PER_TILE = NP // NS
SHIFT = 14


def _scatter_body(u0, u1, pe_hbm, out0, out1,
                  pk_v, idx_v, buf_a, buf_b, sem_a, sem_b, sem_sa, sem_sb,
                  acc):
    c = lax.axis_index("c")
    s = lax.axis_index("s")
    nblk = EP // NS // B

    pltpu.sync_copy(pe_hbm.at[pl.ds(s * nblk, nblk)], pk_v)

    def unpack(j, slot):
        for k in range(B // 16):
            v = pk_v[j, pl.ds(k * 16, 16)]
            idx_v[slot, pl.ds(k * 16, 16)] = v & ((1 << SHIFT) - 1)
            idx_v[2 + slot, pl.ds(k * 16, 16)] = v >> SHIFT

    def zrow(i, _):
        for q in range(HH // 16):
            buf_a[i, pl.ds(q * 16, 16)] = jnp.zeros((16,), jnp.float32)
        return 0

    lax.fori_loop(0, B, zrow, 0)

    def zchunk(k, _):
        pltpu.sync_copy(buf_a, acc.at[pl.ds(s * ROWS_PER_TILE + k * B, B)])
        return 0

    lax.fori_loop(0, ROWS_PER_TILE // B, zchunk, 0)
    plsc.subcore_barrier()

    def run(u_hbm):
        unpack(0, 0)
        pltpu.async_copy(u_hbm.at[idx_v.at[0]], buf_a, sem_a)

        def step2(jj, _):
            j = jj * 2
            unpack(j + 1, 1)
            pltpu.async_copy(u_hbm.at[idx_v.at[1]], buf_b, sem_b)
            pltpu.make_async_copy(u_hbm.at[idx_v.at[0]], buf_a, sem_a).wait()
            pltpu.async_copy(buf_a, acc.at[idx_v.at[2]], sem_sa, add=True)
            pltpu.make_async_copy(u_hbm.at[idx_v.at[1]], buf_b, sem_b).wait()
            pltpu.async_copy(buf_b, acc.at[idx_v.at[3]], sem_sb, add=True)
            pltpu.make_async_copy(buf_a, acc.at[idx_v.at[2]], sem_sa).wait()

            @pl.when(jj < nblk // 2 - 1)
            def _():
                unpack(j + 2, 0)
                pltpu.async_copy(u_hbm.at[idx_v.at[0]], buf_a, sem_a)

            pltpu.make_async_copy(buf_b, acc.at[idx_v.at[3]], sem_sb).wait()
            return 0

        lax.fori_loop(0, nblk // 2, step2, 0)

    @pl.when(c == 0)
    def _():
        run(u0)

    @pl.when(c == 1)
    def _():
        run(u1)

    plsc.subcore_barrier()
    sl = pl.ds(s * ROWS_PER_TILE, ROWS_PER_TILE)

    @pl.when(c == 0)
    def _():
        pltpu.sync_copy(acc.at[sl], out0.at[sl])

    @pl.when(c == 1)
    def _():
        pltpu.sync_copy(acc.at[sl], out1.at[sl])


def _degones_body(pe_hbm, out0, out1, pk_v, idx_v, buf_a, acc):
    c = lax.axis_index("c")
    s = lax.axis_index("s")
    nslab = EP // NS // B
    nblk = nslab // NC

    pltpu.sync_copy(pe_hbm.at[pl.ds(s * nslab, nslab)], pk_v)

    def zrow(i, _):
        for q in range(HH // 16):
            buf_a[i, pl.ds(q * 16, 16)] = jnp.zeros((16,), jnp.float32)
        return 0

    lax.fori_loop(0, B, zrow, 0)

    def zchunk(k, _):
        pltpu.sync_copy(buf_a, acc.at[pl.ds(s * ROWS_PER_TILE + k * B, B)])
        return 0

    lax.fori_loop(0, ROWS_PER_TILE // B, zchunk, 0)

    def orow(i, _):
        for q in range(HH // 16):
            buf_a[i, pl.ds(q * 16, 16)] = jnp.full((16,), 1.0, jnp.float32)
        return 0

    lax.fori_loop(0, B, orow, 0)
    plsc.subcore_barrier()

    def blk(jj, _):
        j = c * nblk + jj
        for k in range(B // 16):
            idx_v[2, pl.ds(k * 16, 16)] = pk_v[j, pl.ds(k * 16, 16)] >> SHIFT
        pltpu.sync_copy(buf_a, acc.at[idx_v.at[2]], add=True)
        return 0

    lax.fori_loop(0, nblk, blk, 0)
    plsc.subcore_barrier()
    sl = pl.ds(s * ROWS_PER_TILE, ROWS_PER_TILE)

    @pl.when(c == 0)
    def _():
        pltpu.sync_copy(acc.at[sl], out0.at[sl])

    @pl.when(c == 1)
    def _():
        pltpu.sync_copy(acc.at[sl], out1.at[sl])


@functools.lru_cache(maxsize=None)
def _sc_kernels():
    mesh = plsc.VectorSubcoreMesh(
        core_axis_name="c", subcore_axis_name="s",
        num_cores=NC, num_subcores=NS)
    scatter = pl.kernel(
        _scatter_body,
        out_type=(
            jax.ShapeDtypeStruct((NP, HH), jnp.float32),
            jax.ShapeDtypeStruct((NP, HH), jnp.float32),
        ),
        mesh=mesh,
        scratch_types=[
            pltpu.VMEM((EP // NS // B, B), jnp.int32),
            pltpu.VMEM((4, B), jnp.int32),
            pltpu.VMEM((B, HH), jnp.float32),
            pltpu.VMEM((B, HH), jnp.float32),
            pltpu.SemaphoreType.DMA,
            pltpu.SemaphoreType.DMA,
            pltpu.SemaphoreType.DMA,
            pltpu.SemaphoreType.DMA,
            pltpu.VMEM_SHARED((NP, HH), jnp.float32),
        ],
    )
    degones = pl.kernel(
        _degones_body,
        out_type=(
            jax.ShapeDtypeStruct((NP, HH), jnp.float32),
            jax.ShapeDtypeStruct((NP, HH), jnp.float32),
        ),
        mesh=mesh,
        scratch_types=[
            pltpu.VMEM((EP // NS // B, B), jnp.int32),
            pltpu.VMEM((4, B), jnp.int32),
            pltpu.VMEM((B, HH), jnp.float32),
            pltpu.VMEM_SHARED((NP, HH), jnp.float32),
        ],
    )
    return degones, scatter


_HI = jax.lax.Precision.HIGHEST


def _dinv_block(d0_ref, d1_ref):
    return lax.rsqrt(1.0 + d0_ref[:, 0:1] + d1_ref[:, 0:1])


def _dense1_body(x_ref, w_ref, d0_ref, d1_ref, u0_ref, u1_ref):
    dinv = _dinv_block(d0_ref, d1_ref)
    h = jnp.dot(x_ref[...], w_ref[...],
                preferred_element_type=jnp.float32, precision=_HI)
    u = h * dinv
    u0_ref[...] = u[:, :HH]
    u1_ref[...] = u[:, HH:]


def _dense2_body(s0_ref, s1_ref, u0_ref, u1_ref, d0_ref, d1_ref, b_ref,
                 w_ref, v0_ref, v1_ref):
    dinv = _dinv_block(d0_ref, d1_ref)
    agg = jnp.concatenate(
        [s0_ref[...] + u0_ref[...], s1_ref[...] + u1_ref[...]], axis=1)
    hin = jnp.maximum(agg * dinv + b_ref[...], 0.0)
    h = jnp.dot(hin, w_ref[...],
                preferred_element_type=jnp.float32, precision=_HI)
    v = h * dinv
    v0_ref[...] = v[:, :HH]
    v1_ref[...] = v[:, HH:]


def _readout_body(s0_ref, s1_ref, u0_ref, u1_ref, d0_ref, d1_ref, b_ref,
                  wr1_ref, br1_ref, wr2_ref, br2_ref, wo_ref, bo_ref,
                  out_ref, acc_ref):
    i = pl.program_id(0)
    dinv = _dinv_block(d0_ref, d1_ref)
    agg = jnp.concatenate(
        [s0_ref[...] + u0_ref[...], s1_ref[...] + u1_ref[...]], axis=1)
    h = jnp.maximum(agg * dinv + b_ref[...], 0.0)
    row = lax.broadcasted_iota(jnp.int32, (R, D), 0) + i * R
    h = jnp.where(row < N, h, 0.0)
    part = jnp.sum(h, axis=0, keepdims=True)

    @pl.when(i == 0)
    def _():
        acc_ref[...] = part

    @pl.when(i > 0)
    def _():
        acc_ref[...] = acc_ref[...] + part

    @pl.when(i == pl.num_programs(0) - 1)
    def _():
        g = acc_ref[...] * (1.0 / N)
        r = jnp.maximum(
            jnp.dot(g, wr1_ref[...], preferred_element_type=jnp.float32,
                    precision=_HI) + br1_ref[...], 0.0)
        r = jnp.maximum(
            jnp.dot(r, wr2_ref[...], preferred_element_type=jnp.float32,
                    precision=_HI) + br2_ref[...], 0.0)
        out_ref[...] = (
            jnp.dot(r, wo_ref[...], preferred_element_type=jnp.float32,
                    precision=_HI) + bo_ref[...])


def _row_spec(width):
    return pl.BlockSpec((R, width), lambda i: (i, 0))


def _full_spec(shape):
    nd = len(shape)
    return pl.BlockSpec(shape, lambda i: (0,) * nd)


_dense1_call = pl.pallas_call(
    _dense1_body,
    grid=(GRID,),
    in_specs=[_row_spec(D), _full_spec((D, D)), _row_spec(HH), _row_spec(HH)],
    out_specs=[_row_spec(HH), _row_spec(HH)],
    out_shape=[jax.ShapeDtypeStruct((NP, HH), jnp.float32)] * 2,
)

_dense2_call = pl.pallas_call(
    _dense2_body,
    grid=(GRID,),
    in_specs=[_row_spec(HH)] * 6 +
             [_full_spec((1, D)), _full_spec((D, D))],
    out_specs=[_row_spec(HH), _row_spec(HH)],
    out_shape=[jax.ShapeDtypeStruct((NP, HH), jnp.float32)] * 2,
)

_readout_call = pl.pallas_call(
    _readout_body,
    grid=(GRID,),
    in_specs=[_row_spec(HH)] * 6 +
             [_full_spec((1, D)),
              _full_spec((D, D)), _full_spec((1, D)),
              _full_spec((D, D)), _full_spec((1, D)),
              _full_spec((D, 1)), _full_spec((1, 1))],
    out_specs=pl.BlockSpec((1, 1), lambda i: (0, 0)),
    out_shape=jax.ShapeDtypeStruct((1, 1), jnp.float32),
    scratch_shapes=[pltpu.VMEM((1, D), jnp.float32)],
)


def kernel(x, edge_index, W1, b1, W2, b2, Wr1, br1, Wr2, br2, Wo, bo):
    src = edge_index[0].astype(jnp.int32)
    dst = edge_index[1].astype(jnp.int32)
    padv = jnp.full((EP - E,), DUMMY, jnp.int32)
    srcp = jnp.concatenate([src, padv])
    dstp = jnp.concatenate([dst, padv])
    packed = srcp | (dstp << SHIFT)
    pe = packed.reshape(EP // B, B)
    xp = jnp.pad(x, ((0, NP - N), (0, 0)))
    b1r = b1.reshape(1, D)
    b2r = b2.reshape(1, D)
    br1r = br1.reshape(1, D)
    br2r = br2.reshape(1, D)
    bor = bo.reshape(1, 1)

    degones_sc, scatter_sc = _sc_kernels()

    d0, d1 = degones_sc(pe)
    u0, u1 = _dense1_call(xp, W1, d0, d1)
    s0, s1 = scatter_sc(u0, u1, pe)
    v0, v1 = _dense2_call(s0, s1, u0, u1, d0, d1, b1r, W2)
    t0, t1 = scatter_sc(v0, v1, pe)
    out = _readout_call(t0, t1, v0, v1, d0, d1, b2r,
                        Wr1, br1r, Wr2, br2r, Wo, bor)
    return out


# speedup vs baseline: 1.0813x; 1.0813x over previous
import functools

import jax
import jax.numpy as jnp
from jax import lax
from jax.experimental import pallas as pl
from jax.experimental.pallas import tpu as pltpu
from jax.experimental.pallas import tpu_sc as plsc

N = 10000
NP = 10240
D = 256
HH = 128
E = 160000
EP = 163840
DUMMY = 10000
NC = 2
NS = 16
B = 128
R = 512
GRID = NP // R
ROWS_PER_TILE = NP // NS
SHIFT = 14


def _scatter_body(u0, u1, pe_hbm, out0, out1,
                  pk_v, idx_v, buf_a, buf_b, sem_a, sem_b, acc):
    c = lax.axis_index("c")
    s = lax.axis_index("s")
    nblk = EP // NS // B

    pltpu.sync_copy(pe_hbm.at[pl.ds(s * nblk, nblk)], pk_v)

    def unpack(j, slot):
        for k in range(B // 16):
            v = pk_v[j, pl.ds(k * 16, 16)]
            idx_v[slot, pl.ds(k * 16, 16)] = v & ((1 << SHIFT) - 1)
            idx_v[2 + slot, pl.ds(k * 16, 16)] = v >> SHIFT

    def zrow(i, _):
        for q in range(HH // 16):
            buf_a[i, pl.ds(q * 16, 16)] = jnp.zeros((16,), jnp.float32)
        return 0

    lax.fori_loop(0, B, zrow, 0)

    def zchunk(k, _):
        pltpu.sync_copy(buf_a, acc.at[pl.ds(s * ROWS_PER_TILE + k * B, B)])
        return 0

    lax.fori_loop(0, ROWS_PER_TILE // B, zchunk, 0)
    plsc.subcore_barrier()

    def run(u_hbm):
        unpack(0, 0)
        pltpu.async_copy(u_hbm.at[idx_v.at[0]], buf_a, sem_a)

        def step2(jj, _):
            j = jj * 2
            unpack(j + 1, 1)
            pltpu.async_copy(u_hbm.at[idx_v.at[1]], buf_b, sem_b)
            pltpu.make_async_copy(u_hbm.at[idx_v.at[0]], buf_a, sem_a).wait()
            pltpu.sync_copy(buf_a, acc.at[idx_v.at[2]], add=True)

            @pl.when(jj < nblk // 2 - 1)
            def _():
                unpack(j + 2, 0)
                pltpu.async_copy(u_hbm.at[idx_v.at[0]], buf_a, sem_a)

            pltpu.make_async_copy(u_hbm.at[idx_v.at[1]], buf_b, sem_b).wait()
            pltpu.sync_copy(buf_b, acc.at[idx_v.at[3]], add=True)
            return 0

        lax.fori_loop(0, nblk // 2, step2, 0)

    @pl.when(c == 0)
    def _():
        run(u0)

    @pl.when(c == 1)
    def _():
        run(u1)

    plsc.subcore_barrier()
    sl = pl.ds(s * ROWS_PER_TILE, ROWS_PER_TILE)

    @pl.when(c == 0)
    def _():
        pltpu.sync_copy(acc.at[sl], out0.at[sl])

    @pl.when(c == 1)
    def _():
        pltpu.sync_copy(acc.at[sl], out1.at[sl])


def _degones_body(pe_hbm, out0, out1, pk_v, idx_v, buf_a, acc):
    c = lax.axis_index("c")
    s = lax.axis_index("s")
    nslab = EP // NS // B
    nblk = nslab // NC

    pltpu.sync_copy(pe_hbm.at[pl.ds(s * nslab, nslab)], pk_v)

    def zrow(i, _):
        for q in range(HH // 16):
            buf_a[i, pl.ds(q * 16, 16)] = jnp.zeros((16,), jnp.float32)
        return 0

    lax.fori_loop(0, B, zrow, 0)

    def zchunk(k, _):
        pltpu.sync_copy(buf_a, acc.at[pl.ds(s * ROWS_PER_TILE + k * B, B)])
        return 0

    lax.fori_loop(0, ROWS_PER_TILE // B, zchunk, 0)

    def orow(i, _):
        for q in range(HH // 16):
            buf_a[i, pl.ds(q * 16, 16)] = jnp.full((16,), 1.0, jnp.float32)
        return 0

    lax.fori_loop(0, B, orow, 0)
    plsc.subcore_barrier()

    def blk(jj, _):
        j = c * nblk + jj
        for k in range(B // 16):
            idx_v[2, pl.ds(k * 16, 16)] = pk_v[j, pl.ds(k * 16, 16)] >> SHIFT
        pltpu.sync_copy(buf_a, acc.at[idx_v.at[2]], add=True)
        return 0

    lax.fori_loop(0, nblk, blk, 0)
    plsc.subcore_barrier()
    sl = pl.ds(s * ROWS_PER_TILE, ROWS_PER_TILE)

    @pl.when(c == 0)
    def _():
        pltpu.sync_copy(acc.at[sl], out0.at[sl])

    @pl.when(c == 1)
    def _():
        pltpu.sync_copy(acc.at[sl], out1.at[sl])


@functools.lru_cache(maxsize=None)
def _sc_kernels():
    mesh = plsc.VectorSubcoreMesh(
        core_axis_name="c", subcore_axis_name="s",
        num_cores=NC, num_subcores=NS)
    scatter = pl.kernel(
        _scatter_body,
        out_type=(
            jax.ShapeDtypeStruct((NP, HH), jnp.float32),
            jax.ShapeDtypeStruct((NP, HH), jnp.float32),
        ),
        mesh=mesh,
        scratch_types=[
            pltpu.VMEM((EP // NS // B, B), jnp.int32),
            pltpu.VMEM((4, B), jnp.int32),
            pltpu.VMEM((B, HH), jnp.float32),
            pltpu.VMEM((B, HH), jnp.float32),
            pltpu.SemaphoreType.DMA,
            pltpu.SemaphoreType.DMA,
            pltpu.VMEM_SHARED((NP, HH), jnp.float32),
        ],
    )
    degones = pl.kernel(
        _degones_body,
        out_type=(
            jax.ShapeDtypeStruct((NP, HH), jnp.float32),
            jax.ShapeDtypeStruct((NP, HH), jnp.float32),
        ),
        mesh=mesh,
        scratch_types=[
            pltpu.VMEM((EP // NS // B, B), jnp.int32),
            pltpu.VMEM((4, B), jnp.int32),
            pltpu.VMEM((B, HH), jnp.float32),
            pltpu.VMEM_SHARED((NP, HH), jnp.float32),
        ],
    )
    return degones, scatter


_HI = jax.lax.Precision.HIGHEST


def _dinv_block(d0_ref, d1_ref):
    return lax.rsqrt(1.0 + d0_ref[:, 0:1] + d1_ref[:, 0:1])


def _dense1_body(x_ref, w_ref, d0_ref, d1_ref, u0_ref, u1_ref):
    dinv = _dinv_block(d0_ref, d1_ref)
    h = jnp.dot(x_ref[...], w_ref[...],
                preferred_element_type=jnp.float32, precision=_HI)
    u = h * dinv
    u0_ref[...] = u[:, :HH]
    u1_ref[...] = u[:, HH:]


def _dense2_body(s0_ref, s1_ref, u0_ref, u1_ref, d0_ref, d1_ref, b_ref,
                 w_ref, v0_ref, v1_ref):
    dinv = _dinv_block(d0_ref, d1_ref)
    agg = jnp.concatenate(
        [s0_ref[...] + u0_ref[...], s1_ref[...] + u1_ref[...]], axis=1)
    hin = jnp.maximum(agg * dinv + b_ref[...], 0.0)
    h = jnp.dot(hin, w_ref[...],
                preferred_element_type=jnp.float32, precision=_HI)
    v = h * dinv
    v0_ref[...] = v[:, :HH]
    v1_ref[...] = v[:, HH:]


def _readout_body(s0_ref, s1_ref, u0_ref, u1_ref, d0_ref, d1_ref, b_ref,
                  wr1_ref, br1_ref, wr2_ref, br2_ref, wo_ref, bo_ref,
                  out_ref, acc_ref):
    i = pl.program_id(0)
    dinv = _dinv_block(d0_ref, d1_ref)
    agg = jnp.concatenate(
        [s0_ref[...] + u0_ref[...], s1_ref[...] + u1_ref[...]], axis=1)
    h = jnp.maximum(agg * dinv + b_ref[...], 0.0)
    row = lax.broadcasted_iota(jnp.int32, (R, D), 0) + i * R
    h = jnp.where(row < N, h, 0.0)
    part = jnp.sum(h, axis=0, keepdims=True)

    @pl.when(i == 0)
    def _():
        acc_ref[...] = part

    @pl.when(i > 0)
    def _():
        acc_ref[...] = acc_ref[...] + part

    @pl.when(i == pl.num_programs(0) - 1)
    def _():
        g = acc_ref[...] * (1.0 / N)
        r = jnp.maximum(
            jnp.dot(g, wr1_ref[...], preferred_element_type=jnp.float32,
                    precision=_HI) + br1_ref[...], 0.0)
        r = jnp.maximum(
            jnp.dot(r, wr2_ref[...], preferred_element_type=jnp.float32,
                    precision=_HI) + br2_ref[...], 0.0)
        out_ref[...] = (
            jnp.dot(r, wo_ref[...], preferred_element_type=jnp.float32,
                    precision=_HI) + bo_ref[...])


def _row_spec(width):
    return pl.BlockSpec((R, width), lambda i: (i, 0))


def _full_spec(shape):
    nd = len(shape)
    return pl.BlockSpec(shape, lambda i: (0,) * nd)


_dense1_call = pl.pallas_call(
    _dense1_body,
    grid=(GRID,),
    in_specs=[_row_spec(D), _full_spec((D, D)), _row_spec(HH), _row_spec(HH)],
    out_specs=[_row_spec(HH), _row_spec(HH)],
    out_shape=[jax.ShapeDtypeStruct((NP, HH), jnp.float32)] * 2,
)

_dense2_call = pl.pallas_call(
    _dense2_body,
    grid=(GRID,),
    in_specs=[_row_spec(HH)] * 6 +
             [_full_spec((1, D)), _full_spec((D, D))],
    out_specs=[_row_spec(HH), _row_spec(HH)],
    out_shape=[jax.ShapeDtypeStruct((NP, HH), jnp.float32)] * 2,
)

_readout_call = pl.pallas_call(
    _readout_body,
    grid=(GRID,),
    in_specs=[_row_spec(HH)] * 6 +
             [_full_spec((1, D)),
              _full_spec((D, D)), _full_spec((1, D)),
              _full_spec((D, D)), _full_spec((1, D)),
              _full_spec((D, 1)), _full_spec((1, 1))],
    out_specs=pl.BlockSpec((1, 1), lambda i: (0, 0)),
    out_shape=jax.ShapeDtypeStruct((1, 1), jnp.float32),
    scratch_shapes=[pltpu.VMEM((1, D), jnp.float32)],
)


def kernel(x, edge_index, W1, b1, W2, b2, Wr1, br1, Wr2, br2, Wo, bo):
    src = edge_index[0].astype(jnp.int32)
    dst = edge_index[1].astype(jnp.int32)
    padv = jnp.full((EP - E,), DUMMY, jnp.int32)
    srcp = jnp.concatenate([src, padv])
    dstp = jnp.concatenate([dst, padv])
    packed = srcp | (dstp << SHIFT)
    pe = packed.reshape(EP // B, B)
    xp = jnp.pad(x, ((0, NP - N), (0, 0)))
    b1r = b1.reshape(1, D)
    b2r = b2.reshape(1, D)
    br1r = br1.reshape(1, D)
    br2r = br2.reshape(1, D)
    bor = bo.reshape(1, 1)

    degones_sc, scatter_sc = _sc_kernels()

    d0, d1 = degones_sc(pe)
    u0, u1 = _dense1_call(xp, W1, d0, d1)
    s0, s1 = scatter_sc(u0, u1, pe)
    v0, v1 = _dense2_call(s0, s1, u0, u1, d0, d1, b1r, W2)
    t0, t1 = scatter_sc(v0, v1, pe)
    out = _readout_call(t0, t1, v0, v1, d0, d1, b2r,
                        Wr1, br1r, Wr2, br2r, Wo, bor)
    return out
